# Initial kernel scaffold; baseline (speedup 1.0000x reference)
import jax
import jax.numpy as jnp
from jax.experimental import pallas as pl


def kernel(positions, batch, edge_indices):
    raise NotImplementedError("write your pallas kernel here")


# speedup vs baseline: 7.4337x; 7.4337x over previous
import functools

import jax
import jax.numpy as jnp
from jax import lax
from jax.experimental import pallas as pl
from jax.experimental.pallas import tpu as pltpu
from jax.experimental.pallas import tpu_sc as plsc

NC = 2
NS = 16
NW = NC * NS
D = 8
B = 2000


def _edge_kernel_body(E, pos_hbm, row_hbm, col_hbm, dist_hbm, vec_hbm,
                      rowv, colv, prow, pcol, distv, vecv, sem):
    epw = E // NW
    wid = lax.axis_index("s") * NC + lax.axis_index("c")

    def chunk_body(ci, carry):
        base = wid * epw + ci * B
        pltpu.sync_copy(row_hbm.at[pl.ds(base, B)], rowv)
        pltpu.sync_copy(col_hbm.at[pl.ds(base, B)], colv)
        cp_r = pltpu.async_copy(pos_hbm.at[rowv], prow, sem)
        cp_c = pltpu.async_copy(pos_hbm.at[colv], pcol, sem)
        cp_r.wait()
        cp_c.wait()

        def lane_body(j, carry2):
            o = j * 16
            lid = o + lax.iota(jnp.int32, 16)
            k0 = jnp.zeros((16,), jnp.int32)
            k1 = jnp.full((16,), 1, jnp.int32)
            k2 = jnp.full((16,), 2, jnp.int32)
            rx = plsc.load_gather(prow, [lid, k0])
            ry = plsc.load_gather(prow, [lid, k1])
            rz = plsc.load_gather(prow, [lid, k2])
            cx = plsc.load_gather(pcol, [lid, k0])
            cy = plsc.load_gather(pcol, [lid, k1])
            cz = plsc.load_gather(pcol, [lid, k2])
            dx = rx - cx
            dy = ry - cy
            dz = rz - cz
            sq = dx * dx + dy * dy + dz * dz
            r16 = rowv[pl.ds(o, 16)]
            c16 = colv[pl.ds(o, 16)]
            sqs = jnp.where(r16 != c16, sq, 1.0)
            ibits = plsc.bitcast(sqs, jnp.int32)
            ibits = 0x5F3759DF - lax.shift_right_logical(ibits, 1)
            y = plsc.bitcast(ibits, jnp.float32)
            nh = sqs * -0.5
            y = y * (1.5 + nh * y * y)
            y = y * (1.5 + nh * y * y)
            y = y * (1.5 + nh * y * y)
            distv[pl.ds(o, 16)] = sq * y
            plsc.store_scatter(vecv, [lid, k0], dx * y)
            plsc.store_scatter(vecv, [lid, k1], dy * y)
            plsc.store_scatter(vecv, [lid, k2], dz * y)
            return carry2

        lax.fori_loop(0, B // 16, lane_body, 0, unroll=False)
        pltpu.sync_copy(distv, dist_hbm.at[pl.ds(base, B)])
        pltpu.sync_copy(vecv, vec_hbm.at[pl.ds(base, B)])
        return carry

    lax.fori_loop(0, epw // B, chunk_body, 0, unroll=False)


@functools.partial(jax.jit, static_argnames=())
def _edge_connect_sc(pos_pad, row, col):
    E = row.shape[0]
    mesh = plsc.VectorSubcoreMesh(core_axis_name="c", subcore_axis_name="s",
                                  num_cores=NC, num_subcores=NS)
    body = functools.partial(_edge_kernel_body, E)
    return pl.kernel(
        body,
        out_type=[
            jax.ShapeDtypeStruct((E,), jnp.float32),
            jax.ShapeDtypeStruct((E, 3), jnp.float32),
        ],
        mesh=mesh,
        compiler_params=pltpu.CompilerParams(needs_layout_passes=False,
                                             use_tc_tiling_on_sc=False),
        scratch_types=[
            pltpu.VMEM((B,), jnp.int32),
            pltpu.VMEM((B,), jnp.int32),
            pltpu.VMEM((B, D), jnp.float32),
            pltpu.VMEM((B, D), jnp.float32),
            pltpu.VMEM((B,), jnp.float32),
            pltpu.VMEM((B, 3), jnp.float32),
            pltpu.SemaphoreType.DMA,
        ],
    )(pos_pad, row, col)


def kernel(positions, batch, edge_indices):
    n = positions.shape[0]
    pos_pad = jnp.concatenate(
        [positions, jnp.zeros((n, D - 3), jnp.float32)], axis=1)
    row = edge_indices[0].astype(jnp.int32)
    col = edge_indices[1].astype(jnp.int32)
    dist, vec = _edge_connect_sc(pos_pad, row, col)
    return (edge_indices, dist, vec)


# speedup vs baseline: 7.6608x; 1.0305x over previous
import functools

import jax
import jax.numpy as jnp
from jax import lax
from jax.experimental import pallas as pl
from jax.experimental.pallas import tpu as pltpu
from jax.experimental.pallas import tpu_sc as plsc

NC = 2
NS = 16
NW = NC * NS
D = 8
B = 2000


def _edge_kernel_body(E, pos_hbm, edge_hbm, dist_hbm, vec_hbm,
                      rowv, colv, prow, pcol, distv, vecv, sem):
    epw = E // NW
    wid = lax.axis_index("s") * NC + lax.axis_index("c")

    def chunk_body(ci, carry):
        base = wid * epw + ci * B
        pltpu.sync_copy(edge_hbm.at[0, pl.ds(base, B)], rowv)
        pltpu.sync_copy(edge_hbm.at[1, pl.ds(base, B)], colv)
        cp_r = pltpu.async_copy(pos_hbm.at[rowv], prow, sem)
        cp_c = pltpu.async_copy(pos_hbm.at[colv], pcol, sem)
        cp_r.wait()
        cp_c.wait()

        def lane_body(j, carry2):
            o = j * 16
            lid = o + lax.iota(jnp.int32, 16)
            k0 = jnp.zeros((16,), jnp.int32)
            k1 = jnp.full((16,), 1, jnp.int32)
            k2 = jnp.full((16,), 2, jnp.int32)
            rx = plsc.load_gather(prow, [lid, k0])
            ry = plsc.load_gather(prow, [lid, k1])
            rz = plsc.load_gather(prow, [lid, k2])
            cx = plsc.load_gather(pcol, [lid, k0])
            cy = plsc.load_gather(pcol, [lid, k1])
            cz = plsc.load_gather(pcol, [lid, k2])
            dx = rx - cx
            dy = ry - cy
            dz = rz - cz
            sq = dx * dx + dy * dy + dz * dz
            r16 = rowv[pl.ds(o, 16)]
            c16 = colv[pl.ds(o, 16)]
            sqs = jnp.where(r16 != c16, sq, 1.0)
            ibits = plsc.bitcast(sqs, jnp.int32)
            ibits = 0x5F3759DF - lax.shift_right_logical(ibits, 1)
            y = plsc.bitcast(ibits, jnp.float32)
            nh = sqs * -0.5
            y = y * (1.5 + nh * y * y)
            y = y * (1.5 + nh * y * y)
            y = y * (1.5 + nh * y * y)
            distv[pl.ds(o, 16)] = sq * y
            plsc.store_scatter(vecv, [lid, k0], dx * y)
            plsc.store_scatter(vecv, [lid, k1], dy * y)
            plsc.store_scatter(vecv, [lid, k2], dz * y)
            return carry2

        lax.fori_loop(0, B // 16, lane_body, 0, unroll=False)
        pltpu.sync_copy(distv, dist_hbm.at[pl.ds(base, B)])
        pltpu.sync_copy(vecv, vec_hbm.at[pl.ds(base, B)])
        return carry

    lax.fori_loop(0, epw // B, chunk_body, 0, unroll=False)


def _edge_connect_sc(positions, edge_indices):
    E = edge_indices.shape[1]
    mesh = plsc.VectorSubcoreMesh(core_axis_name="c", subcore_axis_name="s",
                                  num_cores=NC, num_subcores=NS)
    body = functools.partial(_edge_kernel_body, E)
    return pl.kernel(
        body,
        out_type=[
            jax.ShapeDtypeStruct((E,), jnp.float32),
            jax.ShapeDtypeStruct((E, 3), jnp.float32),
        ],
        mesh=mesh,
        compiler_params=pltpu.CompilerParams(needs_layout_passes=False,
                                             use_tc_tiling_on_sc=False),
        scratch_types=[
            pltpu.VMEM((B,), jnp.int32),
            pltpu.VMEM((B,), jnp.int32),
            pltpu.VMEM((B, D), jnp.float32),
            pltpu.VMEM((B, D), jnp.float32),
            pltpu.VMEM((B,), jnp.float32),
            pltpu.VMEM((B, 3), jnp.float32),
            pltpu.SemaphoreType.DMA,
        ],
    )(positions, edge_indices)


def kernel(positions, batch, edge_indices):
    n = positions.shape[0]
    pos_pad = jnp.concatenate(
        [positions, jnp.zeros((n, D - 3), jnp.float32)], axis=1)
    dist, vec = _edge_connect_sc(pos_pad, edge_indices.astype(jnp.int32))
    return (edge_indices, dist, vec)


# speedup vs baseline: 18.1888x; 2.3743x over previous
import functools

import jax
import jax.numpy as jnp
from jax import lax
from jax.experimental import pallas as pl
from jax.experimental.pallas import tpu as pltpu
from jax.experimental.pallas import tpu_sc as plsc

NC = 2
NS = 16
NW = NC * NS
D = 8
B = 2000


def _edge_kernel_body(E, pos_hbm, edge_hbm, dist_hbm, vx_hbm, vy_hbm, vz_hbm,
                      rowv, colv, prow, pcol, distv, vxv, vyv, vzv, sem):
    epw = E // NW
    wid = lax.axis_index("s") * NC + lax.axis_index("c")

    def chunk_body(ci, carry):
        base = wid * epw + ci * B
        pltpu.sync_copy(edge_hbm.at[0, pl.ds(base, B)], rowv)
        pltpu.sync_copy(edge_hbm.at[1, pl.ds(base, B)], colv)
        cp_r = pltpu.async_copy(pos_hbm.at[rowv], prow, sem)
        cp_c = pltpu.async_copy(pos_hbm.at[colv], pcol, sem)
        cp_r.wait()
        cp_c.wait()

        def lane_body(j, carry2):
            o = j * 16
            lid = o + lax.iota(jnp.int32, 16)
            k0 = jnp.zeros((16,), jnp.int32)
            k1 = jnp.full((16,), 1, jnp.int32)
            k2 = jnp.full((16,), 2, jnp.int32)
            rx = plsc.load_gather(prow, [lid, k0])
            ry = plsc.load_gather(prow, [lid, k1])
            rz = plsc.load_gather(prow, [lid, k2])
            cx = plsc.load_gather(pcol, [lid, k0])
            cy = plsc.load_gather(pcol, [lid, k1])
            cz = plsc.load_gather(pcol, [lid, k2])
            dx = rx - cx
            dy = ry - cy
            dz = rz - cz
            sq = dx * dx + dy * dy + dz * dz
            r16 = rowv[pl.ds(o, 16)]
            c16 = colv[pl.ds(o, 16)]
            sqs = jnp.where(r16 != c16, sq, 1.0)
            ibits = plsc.bitcast(sqs, jnp.int32)
            ibits = 0x5F3759DF - lax.shift_right_logical(ibits, 1)
            y = plsc.bitcast(ibits, jnp.float32)
            nh = sqs * -0.5
            y = y * (1.5 + nh * y * y)
            y = y * (1.5 + nh * y * y)
            y = y * (1.5 + nh * y * y)
            distv[pl.ds(o, 16)] = sq * y
            vxv[pl.ds(o, 16)] = dx * y
            vyv[pl.ds(o, 16)] = dy * y
            vzv[pl.ds(o, 16)] = dz * y
            return carry2

        lax.fori_loop(0, B // 16, lane_body, 0, unroll=False)
        pltpu.sync_copy(distv, dist_hbm.at[pl.ds(base, B)])
        pltpu.sync_copy(vxv, vx_hbm.at[pl.ds(base, B)])
        pltpu.sync_copy(vyv, vy_hbm.at[pl.ds(base, B)])
        pltpu.sync_copy(vzv, vz_hbm.at[pl.ds(base, B)])
        return carry

    lax.fori_loop(0, epw // B, chunk_body, 0, unroll=False)


def _edge_connect_sc(positions, edge_indices):
    E = edge_indices.shape[1]
    mesh = plsc.VectorSubcoreMesh(core_axis_name="c", subcore_axis_name="s",
                                  num_cores=NC, num_subcores=NS)
    body = functools.partial(_edge_kernel_body, E)
    return pl.kernel(
        body,
        out_type=[
            jax.ShapeDtypeStruct((E,), jnp.float32),
            jax.ShapeDtypeStruct((E,), jnp.float32),
            jax.ShapeDtypeStruct((E,), jnp.float32),
            jax.ShapeDtypeStruct((E,), jnp.float32),
        ],
        mesh=mesh,
        compiler_params=pltpu.CompilerParams(needs_layout_passes=False,
                                             use_tc_tiling_on_sc=False),
        scratch_types=[
            pltpu.VMEM((B,), jnp.int32),
            pltpu.VMEM((B,), jnp.int32),
            pltpu.VMEM((B, D), jnp.float32),
            pltpu.VMEM((B, D), jnp.float32),
            pltpu.VMEM((B,), jnp.float32),
            pltpu.VMEM((B,), jnp.float32),
            pltpu.VMEM((B,), jnp.float32),
            pltpu.VMEM((B,), jnp.float32),
            pltpu.SemaphoreType.DMA,
        ],
    )(positions, edge_indices)


def kernel(positions, batch, edge_indices):
    n = positions.shape[0]
    pos_pad = jnp.concatenate(
        [positions, jnp.zeros((n, D - 3), jnp.float32)], axis=1)
    dist, vx, vy, vz = _edge_connect_sc(pos_pad, edge_indices.astype(jnp.int32))
    vec = jnp.stack([vx, vy, vz], axis=1)
    return (edge_indices, dist, vec)


# speedup vs baseline: 18.8691x; 1.0374x over previous
import functools

import jax
import jax.numpy as jnp
from jax import lax
from jax.experimental import pallas as pl
from jax.experimental.pallas import tpu as pltpu
from jax.experimental.pallas import tpu_sc as plsc

NC = 2
NS = 16
NW = NC * NS
D = 8
B = 2000


def _edge_kernel_body(E, pos_hbm, edge_hbm, dist_hbm, vx_hbm, vy_hbm, vz_hbm,
                      rowv, colv, prow, pcol, distv, vxv, vyv, vzv, sem):
    epw = E // NW
    wid = lax.axis_index("s") * NC + lax.axis_index("c")

    def chunk_body(ci, carry):
        base = wid * epw + ci * B
        pltpu.sync_copy(edge_hbm.at[0, pl.ds(base, B)], rowv)
        pltpu.sync_copy(edge_hbm.at[1, pl.ds(base, B)], colv)
        cp_r = pltpu.async_copy(pos_hbm.at[rowv], prow, sem)
        cp_c = pltpu.async_copy(pos_hbm.at[colv], pcol, sem)
        cp_r.wait()
        cp_c.wait()

        def lane_body(j, carry2):
            o = j * 16
            lid = o + lax.iota(jnp.int32, 16)
            k0 = jnp.zeros((16,), jnp.int32)
            k1 = jnp.full((16,), 1, jnp.int32)
            k2 = jnp.full((16,), 2, jnp.int32)
            rx = plsc.load_gather(prow, [lid, k0])
            ry = plsc.load_gather(prow, [lid, k1])
            rz = plsc.load_gather(prow, [lid, k2])
            cx = plsc.load_gather(pcol, [lid, k0])
            cy = plsc.load_gather(pcol, [lid, k1])
            cz = plsc.load_gather(pcol, [lid, k2])
            dx = rx - cx
            dy = ry - cy
            dz = rz - cz
            sq = dx * dx + dy * dy + dz * dz
            r16 = rowv[pl.ds(o, 16)]
            c16 = colv[pl.ds(o, 16)]
            sqs = jnp.where(r16 != c16, sq, 1.0)
            ibits = plsc.bitcast(sqs, jnp.int32)
            ibits = 0x5F3759DF - lax.shift_right_logical(ibits, 1)
            y = plsc.bitcast(ibits, jnp.float32)
            nh = sqs * -0.5
            y = y * (1.5 + nh * y * y)
            y = y * (1.5 + nh * y * y)
            distv[pl.ds(o, 16)] = sq * y
            vxv[pl.ds(o, 16)] = dx * y
            vyv[pl.ds(o, 16)] = dy * y
            vzv[pl.ds(o, 16)] = dz * y
            return carry2

        lax.fori_loop(0, B // 16, lane_body, 0, unroll=5)
        pltpu.sync_copy(distv, dist_hbm.at[pl.ds(base, B)])
        pltpu.sync_copy(vxv, vx_hbm.at[pl.ds(base, B)])
        pltpu.sync_copy(vyv, vy_hbm.at[pl.ds(base, B)])
        pltpu.sync_copy(vzv, vz_hbm.at[pl.ds(base, B)])
        return carry

    lax.fori_loop(0, epw // B, chunk_body, 0, unroll=False)


def _edge_connect_sc(positions, edge_indices):
    E = edge_indices.shape[1]
    mesh = plsc.VectorSubcoreMesh(core_axis_name="c", subcore_axis_name="s",
                                  num_cores=NC, num_subcores=NS)
    body = functools.partial(_edge_kernel_body, E)
    return pl.kernel(
        body,
        out_type=[
            jax.ShapeDtypeStruct((E,), jnp.float32),
            jax.ShapeDtypeStruct((E,), jnp.float32),
            jax.ShapeDtypeStruct((E,), jnp.float32),
            jax.ShapeDtypeStruct((E,), jnp.float32),
        ],
        mesh=mesh,
        compiler_params=pltpu.CompilerParams(needs_layout_passes=False,
                                             use_tc_tiling_on_sc=False),
        scratch_types=[
            pltpu.VMEM((B,), jnp.int32),
            pltpu.VMEM((B,), jnp.int32),
            pltpu.VMEM((B, D), jnp.float32),
            pltpu.VMEM((B, D), jnp.float32),
            pltpu.VMEM((B,), jnp.float32),
            pltpu.VMEM((B,), jnp.float32),
            pltpu.VMEM((B,), jnp.float32),
            pltpu.VMEM((B,), jnp.float32),
            pltpu.SemaphoreType.DMA,
        ],
    )(positions, edge_indices)


def kernel(positions, batch, edge_indices):
    n = positions.shape[0]
    pos_pad = jnp.concatenate(
        [positions, jnp.zeros((n, D - 3), jnp.float32)], axis=1)
    dist, vx, vy, vz = _edge_connect_sc(pos_pad, edge_indices.astype(jnp.int32))
    vec = jnp.stack([vx, vy, vz], axis=1)
    return (edge_indices, dist, vec)


# speedup vs baseline: 26.2216x; 1.3897x over previous
import functools

import jax
import jax.numpy as jnp
from jax import lax
from jax.experimental import pallas as pl
from jax.experimental.pallas import tpu as pltpu
from jax.experimental.pallas import tpu_sc as plsc

NC = 2
NS = 16
NW = NC * NS
D = 8
B = 2000


def _edge_kernel_body(E, pos_hbm, edge_hbm, dist_hbm, vx_hbm, vy_hbm, vz_hbm,
                      row0, row1, col0, col1, prow0, prow1, pcol0, pcol1,
                      dist0, dist1, vx0, vx1, vy0, vy1, vz0, vz1,
                      isem0, isem1, gsem0, gsem1, osem0, osem1):
    epw = E // NW
    nch = epw // B
    wid = lax.axis_index("s") * NC + lax.axis_index("c")

    rows = (row0, row1)
    cols = (col0, col1)
    prows = (prow0, prow1)
    pcols = (pcol0, pcol1)
    dists = (dist0, dist1)
    vxs = (vx0, vx1)
    vys = (vy0, vy1)
    vzs = (vz0, vz1)
    isems = (isem0, isem1)
    gsems = (gsem0, gsem1)
    osems = (osem0, osem1)

    def idx_start(ci, s):
        base = wid * epw + ci * B
        pltpu.async_copy(edge_hbm.at[0, pl.ds(base, B)], rows[s], isems[s])
        pltpu.async_copy(edge_hbm.at[1, pl.ds(base, B)], cols[s], isems[s])

    def idx_wait(s):
        pltpu.make_async_copy(edge_hbm.at[0, pl.ds(0, B)], rows[s], isems[s]).wait()
        pltpu.make_async_copy(edge_hbm.at[1, pl.ds(0, B)], cols[s], isems[s]).wait()

    def g_start(s):
        pltpu.async_copy(pos_hbm.at[rows[s]], prows[s], gsems[s])
        pltpu.async_copy(pos_hbm.at[cols[s]], pcols[s], gsems[s])

    def g_wait(s):
        pltpu.make_async_copy(pos_hbm.at[rows[s]], prows[s], gsems[s]).wait()
        pltpu.make_async_copy(pos_hbm.at[cols[s]], pcols[s], gsems[s]).wait()

    def out_start(ci, s):
        base = wid * epw + ci * B
        pltpu.async_copy(dists[s], dist_hbm.at[pl.ds(base, B)], osems[s])
        pltpu.async_copy(vxs[s], vx_hbm.at[pl.ds(base, B)], osems[s])
        pltpu.async_copy(vys[s], vy_hbm.at[pl.ds(base, B)], osems[s])
        pltpu.async_copy(vzs[s], vz_hbm.at[pl.ds(base, B)], osems[s])

    def out_wait(s):
        pltpu.make_async_copy(dists[s], dist_hbm.at[pl.ds(0, B)], osems[s]).wait()
        pltpu.make_async_copy(vxs[s], vx_hbm.at[pl.ds(0, B)], osems[s]).wait()
        pltpu.make_async_copy(vys[s], vy_hbm.at[pl.ds(0, B)], osems[s]).wait()
        pltpu.make_async_copy(vzs[s], vz_hbm.at[pl.ds(0, B)], osems[s]).wait()

    def compute(s):
        rv, cv = rows[s], cols[s]
        pr, pc = prows[s], pcols[s]
        dv, xv, yv, zv = dists[s], vxs[s], vys[s], vzs[s]

        def lane_body(j, carry2):
            o = j * 16
            lid = o + lax.iota(jnp.int32, 16)
            k0 = jnp.zeros((16,), jnp.int32)
            k1 = jnp.full((16,), 1, jnp.int32)
            k2 = jnp.full((16,), 2, jnp.int32)
            rx = plsc.load_gather(pr, [lid, k0])
            ry = plsc.load_gather(pr, [lid, k1])
            rz = plsc.load_gather(pr, [lid, k2])
            cx = plsc.load_gather(pc, [lid, k0])
            cy = plsc.load_gather(pc, [lid, k1])
            cz = plsc.load_gather(pc, [lid, k2])
            dx = rx - cx
            dy = ry - cy
            dz = rz - cz
            sq = dx * dx + dy * dy + dz * dz
            r16 = rv[pl.ds(o, 16)]
            c16 = cv[pl.ds(o, 16)]
            sqs = jnp.where(r16 != c16, sq, 1.0)
            ibits = plsc.bitcast(sqs, jnp.int32)
            ibits = 0x5F3759DF - lax.shift_right_logical(ibits, 1)
            y = plsc.bitcast(ibits, jnp.float32)
            nh = sqs * -0.5
            y = y * (1.5 + nh * y * y)
            y = y * (1.5 + nh * y * y)
            dv[pl.ds(o, 16)] = sq * y
            xv[pl.ds(o, 16)] = dx * y
            yv[pl.ds(o, 16)] = dy * y
            zv[pl.ds(o, 16)] = dz * y
            return carry2

        lax.fori_loop(0, B // 16, lane_body, 0, unroll=5)

    idx_start(0, 0)
    idx_wait(0)
    g_start(0)
    idx_start(1, 1)

    @pl.loop(0, nch + 1, step=2)
    def _chunks(k):
        for s in (0, 1):
            ci = k + s

            @pl.when(ci < nch)
            def _step():
                @pl.when(ci + 1 < nch)
                def _prefetch_gather():
                    idx_wait(1 - s)
                    g_start(1 - s)

                g_wait(s)

                @pl.when(ci >= 2)
                def _drain_out():
                    out_wait(s)

                compute(s)
                out_start(ci, s)

                @pl.when(ci + 2 < nch)
                def _prefetch_idx():
                    idx_start(ci + 2, s)

    out_wait(1 - (nch - 1) % 2)
    out_wait((nch - 1) % 2)


def _edge_connect_sc(positions, edge_indices):
    E = edge_indices.shape[1]
    mesh = plsc.VectorSubcoreMesh(core_axis_name="c", subcore_axis_name="s",
                                  num_cores=NC, num_subcores=NS)
    body = functools.partial(_edge_kernel_body, E)
    return pl.kernel(
        body,
        out_type=[
            jax.ShapeDtypeStruct((E,), jnp.float32),
            jax.ShapeDtypeStruct((E,), jnp.float32),
            jax.ShapeDtypeStruct((E,), jnp.float32),
            jax.ShapeDtypeStruct((E,), jnp.float32),
        ],
        mesh=mesh,
        compiler_params=pltpu.CompilerParams(needs_layout_passes=False,
                                             use_tc_tiling_on_sc=False),
        scratch_types=[
            pltpu.VMEM((B,), jnp.int32),
            pltpu.VMEM((B,), jnp.int32),
            pltpu.VMEM((B,), jnp.int32),
            pltpu.VMEM((B,), jnp.int32),
            pltpu.VMEM((B, D), jnp.float32),
            pltpu.VMEM((B, D), jnp.float32),
            pltpu.VMEM((B, D), jnp.float32),
            pltpu.VMEM((B, D), jnp.float32),
            pltpu.VMEM((B,), jnp.float32),
            pltpu.VMEM((B,), jnp.float32),
            pltpu.VMEM((B,), jnp.float32),
            pltpu.VMEM((B,), jnp.float32),
            pltpu.VMEM((B,), jnp.float32),
            pltpu.VMEM((B,), jnp.float32),
            pltpu.VMEM((B,), jnp.float32),
            pltpu.VMEM((B,), jnp.float32),
            pltpu.SemaphoreType.DMA,
            pltpu.SemaphoreType.DMA,
            pltpu.SemaphoreType.DMA,
            pltpu.SemaphoreType.DMA,
            pltpu.SemaphoreType.DMA,
            pltpu.SemaphoreType.DMA,
        ],
    )(positions, edge_indices)


def kernel(positions, batch, edge_indices):
    n = positions.shape[0]
    pos_pad = jnp.concatenate(
        [positions, jnp.zeros((n, D - 3), jnp.float32)], axis=1)
    dist, vx, vy, vz = _edge_connect_sc(pos_pad, edge_indices.astype(jnp.int32))
    vec = jnp.stack([vx, vy, vz], axis=1)
    return (edge_indices, dist, vec)
